# baseline (speedup 1.0000x reference)
import functools

import jax
import jax.numpy as jnp
from jax import lax
from jax.experimental import pallas as pl
from jax.experimental.pallas import tpu as pltpu
from jax.experimental.pallas import tpu_sc as plsc

_B, _T = 8, 4
_H, _W = 48, 48
_C = 768
_NC = 2
_LANES = 16
_NW = 32


def _sc_copy(meta, emb, zrow):
    mesh = plsc.VectorSubcoreMesh(core_axis_name="c", subcore_axis_name="s")

    @functools.partial(
        pl.kernel,
        mesh=mesh,
        out_type=jax.ShapeDtypeStruct((_B, _T, _H, _W, _C), jnp.float32),
        scratch_types=[
            pltpu.VMEM((4 * _LANES,), jnp.int32),
            pltpu.VMEM((3, _W, _C), jnp.float32),
            pltpu.SemaphoreType.DMA,
            pltpu.SemaphoreType.DMA,
        ],
    )
    def k(meta_hbm, emb_hbm, z_hbm, out_hbm, meta_v, buf, sem_in, sem_out):
        wid = lax.axis_index("s") * _NC + lax.axis_index("c")
        pltpu.sync_copy(meta_hbm.at[wid], meta_v)
        mv = meta_v[pl.ds(0, _LANES)]
        bl0 = meta_v[pl.ds(2 * _LANES, _LANES)]
        bl1 = meta_v[pl.ds(3 * _LANES, _LANES)]
        sel = mv[0]
        h0 = mv[1]
        g = mv[2]
        gpos = mv[3]
        bs = mv[4]

        def wait_in():
            pltpu.make_async_copy(z_hbm, buf.at[0], sem_in).wait()

        def wait_out():
            pltpu.make_async_copy(z_hbm, buf.at[0], sem_out).wait()

        def start_in(k_, h):
            pltpu.async_copy(
                emb_hbm.at[sel, h0 + h, pl.ds(0, _W), :],
                buf.at[lax.rem(k_, 3)],
                sem_in,
            )

        def fan_out(k_, h):
            for j in range(_NW):
                lane = bl0[j] if j < _LANES else bl1[j - _LANES]

                @pl.when(j < bs)
                def _(bid=lane):
                    bb = bid // _T
                    tt = bid - bb * _T
                    pltpu.async_copy(
                        buf.at[lax.rem(k_, 3)], out_hbm.at[bb, tt, h], sem_out
                    )

        @pl.when(sel >= 0)
        def _copy():
            m = (_H - 1 - gpos) // g + 1

            start_in(0, gpos)

            @pl.when(m >= 2)
            def _pre2():
                start_in(1, gpos + g)

            def body(k_, carry):
                h = gpos + k_ * g
                wait_in()
                fan_out(k_, h)

                @pl.when(k_ + 2 < m)
                def _more():
                    @pl.when(k_ >= 1)
                    def _free():
                        lax.fori_loop(
                            0, bs, lambda i, c: (wait_out(), c)[1], 0
                        )

                    start_in(k_ + 2, h + 2 * g)

                return carry

            lax.fori_loop(0, m, body, 0)
            rem = (m - jnp.maximum(m - 3, 0)) * bs
            lax.fori_loop(0, rem, lambda i, c: (wait_out(), c)[1], 0)

        @pl.when(sel < 0)
        def _zero():
            pltpu.sync_copy(z_hbm, buf.at[0])
            b = wid // _T
            t = wid - b * _T

            def fire(h, carry):
                pltpu.async_copy(buf.at[0], out_hbm.at[b, t, h], sem_out)
                return carry

            lax.fori_loop(0, _H, fire, 0)

            def drain(h, carry):
                wait_out()
                return carry

            lax.fori_loop(0, _H, drain, 0)

    return k(meta, emb, zrow)


def kernel(time_constant, height, width, emb):
    tc = time_constant.astype(jnp.int32).reshape(-1)
    h0 = (jnp.asarray(height, jnp.int32) - _H).astype(jnp.int32)
    n = tc.shape[0]
    sel = jnp.where(tc > 0, jnp.abs(tc), jnp.int32(-1))

    w = jnp.arange(n, dtype=jnp.int32)
    vals = jnp.arange(10, dtype=jnp.int32)
    hit = (sel[:, None] == vals[None, :]) & (sel[:, None] >= 0)
    cnt = hit.sum(axis=0).astype(jnp.int32)
    sel_c = jnp.maximum(sel, 0)
    rank = (jnp.cumsum(hit.astype(jnp.int32), axis=0) - 1)[w, sel_c]
    sel_row = jnp.where(sel >= 0, sel, jnp.int32(10))
    blist = jnp.zeros((11, _NW), jnp.int32).at[sel_row, rank].set(w)

    nzm = sel >= 0
    nzt = nzm.sum().astype(jnp.float32)
    present = cnt > 0
    wgt = jnp.where(present, 1 + cnt, 0).astype(jnp.float32)
    ideal = nzt * wgt / jnp.maximum(wgt.sum(), 1.0)
    rs0 = jnp.where(
        present, jnp.maximum(jnp.floor(ideal).astype(jnp.int32), 1), 0
    )
    deficit = nzt.astype(jnp.int32) - rs0.sum()
    frac = jnp.where(present, ideal - jnp.floor(ideal), -1.0)
    frk = jnp.argsort(jnp.argsort(-frac)).astype(jnp.int32)
    rs = rs0 + (present & (frk < deficit)).astype(jnp.int32)
    cum = jnp.cumsum(rs)
    start = cum - rs
    nzrank = jnp.cumsum(nzm.astype(jnp.int32)) - 1
    v_srv = (nzrank[:, None] >= cum[None, :]).sum(axis=1).astype(jnp.int32)
    v_srv = jnp.minimum(v_srv, 9)
    sel_srv = jnp.where(nzm, v_srv, jnp.int32(-1))
    g_srv = jnp.where(nzm, jnp.maximum(rs[v_srv], 1), 1)
    gpos = jnp.where(nzm, nzrank - start[v_srv], 0)
    bs_srv = jnp.where(nzm, cnt[v_srv], 0)
    rows = blist[v_srv]

    head = jnp.stack(
        [sel_srv, jnp.broadcast_to(h0, (n,)), g_srv, gpos, bs_srv], axis=1
    )
    head = jnp.pad(head, ((0, 0), (0, 2 * _LANES - head.shape[1])))
    meta = jnp.concatenate([head, rows], axis=1)
    zrow = jnp.zeros((_W, _C), jnp.float32)
    return _sc_copy(meta.astype(jnp.int32), emb, zrow)


# speedup vs baseline: 1.0946x; 1.0946x over previous
import functools

import jax
import jax.numpy as jnp
from jax import lax
from jax.experimental import pallas as pl
from jax.experimental.pallas import tpu as pltpu
from jax.experimental.pallas import tpu_sc as plsc

_B, _T = 8, 4
_H, _W = 48, 48
_C = 768
_NC = 2
_LANES = 16
_NW = 32


def _sc_copy(meta, emb, zrow):
    mesh = plsc.VectorSubcoreMesh(core_axis_name="c", subcore_axis_name="s")

    @functools.partial(
        pl.kernel,
        mesh=mesh,
        out_type=jax.ShapeDtypeStruct((_B, _T, _H, _W, _C), jnp.float32),
        scratch_types=[
            pltpu.VMEM((4 * _LANES,), jnp.int32),
            pltpu.VMEM((3, _W, _C), jnp.float32),
            pltpu.SemaphoreType.DMA,
            pltpu.SemaphoreType.DMA,
        ],
    )
    def k(meta_hbm, emb_hbm, z_hbm, out_hbm, meta_v, buf, sem_in, sem_out):
        wid = lax.axis_index("s") * _NC + lax.axis_index("c")
        pltpu.sync_copy(meta_hbm.at[wid], meta_v)
        mv = meta_v[pl.ds(0, _LANES)]
        bl0 = meta_v[pl.ds(2 * _LANES, _LANES)]
        bl1 = meta_v[pl.ds(3 * _LANES, _LANES)]
        sel = mv[0]
        h0 = mv[1]
        g = mv[2]
        gpos = mv[3]

        def wait_in():
            pltpu.make_async_copy(z_hbm, buf.at[0], sem_in).wait()

        def wait_out():
            pltpu.make_async_copy(z_hbm, buf.at[0], sem_out).wait()

        def start_in(k_, h):
            pltpu.async_copy(
                emb_hbm.at[sel, h0 + h, pl.ds(0, _W), :],
                buf.at[lax.rem(k_, 3)],
                sem_in,
            )

        def fan_out(k_, h):
            for j in range(_NW):
                lane = bl0[j] if j < _LANES else bl1[j - _LANES]

                @pl.when(j < g)
                def _(bid=lane):
                    bb = bid // _T
                    tt = bid - bb * _T
                    pltpu.async_copy(
                        buf.at[lax.rem(k_, 3)], out_hbm.at[bb, tt, h], sem_out
                    )

        @pl.when(sel >= 0)
        def _copy():
            m = (_H - 1 - gpos) // g + 1

            start_in(0, gpos)

            @pl.when(m >= 2)
            def _pre2():
                start_in(1, gpos + g)

            def body(k_, carry):
                h = gpos + k_ * g
                wait_in()
                fan_out(k_, h)

                @pl.when(k_ + 2 < m)
                def _more():
                    @pl.when(k_ >= 1)
                    def _free():
                        lax.fori_loop(
                            0, g, lambda i, c: (wait_out(), c)[1], 0
                        )

                    start_in(k_ + 2, h + 2 * g)

                return carry

            lax.fori_loop(0, m, body, 0)
            rem = (m - jnp.maximum(m - 3, 0)) * g
            lax.fori_loop(0, rem, lambda i, c: (wait_out(), c)[1], 0)

        @pl.when(sel < 0)
        def _zero():
            pltpu.sync_copy(z_hbm, buf.at[0])
            b = wid // _T
            t = wid - b * _T

            def fire(h, carry):
                pltpu.async_copy(buf.at[0], out_hbm.at[b, t, h], sem_out)
                return carry

            lax.fori_loop(0, _H, fire, 0)

            def drain(h, carry):
                wait_out()
                return carry

            lax.fori_loop(0, _H, drain, 0)

    return k(meta, emb, zrow)


def kernel(time_constant, height, width, emb):
    tc = time_constant.astype(jnp.int32).reshape(-1)
    h0 = (jnp.asarray(height, jnp.int32) - _H).astype(jnp.int32)
    n = tc.shape[0]
    sel = jnp.where(tc > 0, jnp.abs(tc), jnp.int32(-1))

    w = jnp.arange(n, dtype=jnp.int32)
    vals = jnp.arange(10, dtype=jnp.int32)
    hit = (sel[:, None] == vals[None, :]) & (sel[:, None] >= 0)
    cnt = hit.sum(axis=0).astype(jnp.int32)
    sel_c = jnp.maximum(sel, 0)
    rank = (jnp.cumsum(hit.astype(jnp.int32), axis=0) - 1)[w, sel_c]
    g = cnt[sel_c]
    sel_row = jnp.where(sel >= 0, sel, jnp.int32(10))
    blist = jnp.zeros((11, _NW), jnp.int32).at[sel_row, rank].set(w)
    rows = blist[sel_c]

    head = jnp.stack(
        [sel, jnp.broadcast_to(h0, (n,)), g, rank], axis=1
    )
    head = jnp.pad(head, ((0, 0), (0, 2 * _LANES - head.shape[1])))
    meta = jnp.concatenate([head, rows], axis=1)
    zrow = jnp.zeros((_W, _C), jnp.float32)
    return _sc_copy(meta.astype(jnp.int32), emb, zrow)
